# baseline (speedup 1.0000x reference)
import functools

import jax
import jax.numpy as jnp
from jax import lax
from jax.experimental import pallas as pl
from jax.experimental.pallas import tpu as pltpu
from jax.experimental.pallas import tpu_sc as plsc

_B, _S, _D = 4, 8192, 128
_NC, _NS, _L = 2, 16, 16
_NW = _NC * _NS
_P = _S // _NW
_RSUB = 16
_NSUB = _P // _RSUB
_NSLOT = 8
_LEAD = 7


def _body(we_hbm, pos_hbm, out_hbm, pos_v, *scratch):
    buf = scratch[0:_NSLOT]
    psem = scratch[_NSLOT:_NSLOT + 2]
    ld = scratch[_NSLOT + 2:_NSLOT + 2 + _NSLOT]
    st = scratch[_NSLOT + 2 + _NSLOT:_NSLOT + 2 + 2 * _NSLOT]

    wid = lax.axis_index("s") * _NC + lax.axis_index("c")
    prow = wid * _P

    _PHEAD = _LEAD * _RSUB
    pos_a = pltpu.make_async_copy(
        pos_hbm.at[pl.ds(prow, _PHEAD)], pos_v.at[pl.ds(0, _PHEAD)], psem[0])
    pos_b = pltpu.make_async_copy(
        pos_hbm.at[pl.ds(prow + _PHEAD, _P - _PHEAD)],
        pos_v.at[pl.ds(_PHEAD, _P - _PHEAD)], psem[1])
    pos_a.start()
    pos_b.start()

    def fire_loads(t):
        slot = t % _NSLOT
        r0 = prow + t * _RSUB
        pltpu.async_copy(
            we_hbm.at[:, pl.ds(r0, _RSUB)], buf[slot], ld[slot])

    def wait_loads(t):
        slot = t % _NSLOT
        r0 = prow + t * _RSUB
        pltpu.make_async_copy(
            we_hbm.at[:, pl.ds(r0, _RSUB)], buf[slot], ld[slot]).wait()

    def fire_stores(t):
        slot = t % _NSLOT
        r0 = prow + t * _RSUB
        pltpu.async_copy(
            buf[slot], out_hbm.at[:, pl.ds(r0, _RSUB)], st[slot])

    def wait_stores(t):
        slot = t % _NSLOT
        r0 = prow + t * _RSUB
        pltpu.make_async_copy(
            buf[slot], out_hbm.at[:, pl.ds(r0, _RSUB)], st[slot]).wait()

    for t in range(_LEAD):
        fire_loads(t)
    pos_a.wait()

    for t in range(_NSUB):
        slot = t % _NSLOT
        wait_loads(t)
        if t == _LEAD:
            pos_b.wait()

        def row(i, carry):
            for j in range(_D // _L):
                sl = pl.ds(j * _L, _L)
                p = pos_v[t * _RSUB + i, sl]
                for b in range(_B):
                    buf[slot][b, i, sl] = buf[slot][b, i, sl] + p
            return carry

        lax.fori_loop(0, _RSUB, row, 0)
        fire_stores(t)
        if t + _LEAD < _NSUB:
            if t + _LEAD >= _NSLOT:
                wait_stores(t + _LEAD - _NSLOT)
            fire_loads(t + _LEAD)

    for t in range(_NSUB - _NSLOT, _NSUB):
        if t >= 0:
            wait_stores(t)


@jax.jit
def _sc_add(we, pos):
    mesh = plsc.VectorSubcoreMesh(core_axis_name="c", subcore_axis_name="s")
    f = functools.partial(
        pl.kernel,
        out_type=jax.ShapeDtypeStruct((_B, _S, _D), jnp.float32),
        mesh=mesh,
        scratch_types=(
            [pltpu.VMEM((_P, _D), jnp.float32)]
            + [pltpu.VMEM((_B, _RSUB, _D), jnp.float32)
               for _ in range(_NSLOT)]
            + [pltpu.SemaphoreType.DMA for _ in range(2 * _NSLOT + 2)]
        ),
    )(_body)
    return f(we, pos)


def kernel(input_ids, word_embeddings, pos_table):
    del input_ids
    return _sc_add(word_embeddings, pos_table)


# speedup vs baseline: 1.0378x; 1.0378x over previous
import functools

import jax
import jax.numpy as jnp
from jax import lax
from jax.experimental import pallas as pl
from jax.experimental.pallas import tpu as pltpu
from jax.experimental.pallas import tpu_sc as plsc

_B, _S, _D = 4, 8192, 128
_NC, _NS, _L = 2, 16, 16
_NW = _NC * _NS
_P = _S // _NW
_RSUB = 32
_NSUB = _P // _RSUB
_NSLOT = 5
_LEAD = 4


def _body(we_hbm, pos_hbm, out_hbm, pos_v, *scratch):
    buf = scratch[0:_NSLOT]
    psem = scratch[_NSLOT:_NSLOT + 2]
    ld = scratch[_NSLOT + 2:_NSLOT + 2 + _NSLOT]
    st = scratch[_NSLOT + 2 + _NSLOT:_NSLOT + 2 + 2 * _NSLOT]

    wid = lax.axis_index("c") * _NS + lax.axis_index("s")
    prow = wid * _P

    _PHEAD = _LEAD * _RSUB
    pos_a = pltpu.make_async_copy(
        pos_hbm.at[pl.ds(prow, _PHEAD)], pos_v.at[pl.ds(0, _PHEAD)], psem[0])
    pos_b = pltpu.make_async_copy(
        pos_hbm.at[pl.ds(prow + _PHEAD, _P - _PHEAD)],
        pos_v.at[pl.ds(_PHEAD, _P - _PHEAD)], psem[1])
    pos_a.start()
    pos_b.start()

    def fire_loads(t):
        slot = t % _NSLOT
        r0 = prow + t * _RSUB
        pltpu.async_copy(
            we_hbm.at[:, pl.ds(r0, _RSUB)], buf[slot], ld[slot])

    def wait_loads(t):
        slot = t % _NSLOT
        r0 = prow + t * _RSUB
        pltpu.make_async_copy(
            we_hbm.at[:, pl.ds(r0, _RSUB)], buf[slot], ld[slot]).wait()

    def fire_stores(t):
        slot = t % _NSLOT
        r0 = prow + t * _RSUB
        pltpu.async_copy(
            buf[slot], out_hbm.at[:, pl.ds(r0, _RSUB)], st[slot])

    def wait_stores(t):
        slot = t % _NSLOT
        r0 = prow + t * _RSUB
        pltpu.make_async_copy(
            buf[slot], out_hbm.at[:, pl.ds(r0, _RSUB)], st[slot]).wait()

    for t in range(_LEAD):
        fire_loads(t)
    pos_a.wait()

    for t in range(_NSUB):
        slot = t % _NSLOT
        wait_loads(t)
        if t == _LEAD:
            pos_b.wait()

        def row(i, carry):
            for j in range(_D // _L):
                sl = pl.ds(j * _L, _L)
                p = pos_v[t * _RSUB + i, sl]
                for b in range(_B):
                    buf[slot][b, i, sl] = buf[slot][b, i, sl] + p
            return carry

        lax.fori_loop(0, _RSUB, row, 0)
        fire_stores(t)
        if t + _LEAD < _NSUB:
            if t + _LEAD >= _NSLOT:
                wait_stores(t + _LEAD - _NSLOT)
            fire_loads(t + _LEAD)

    for t in range(_NSUB - _NSLOT, _NSUB):
        if t >= 0:
            wait_stores(t)


@jax.jit
def _sc_add(we, pos):
    mesh = plsc.VectorSubcoreMesh(core_axis_name="c", subcore_axis_name="s")
    f = functools.partial(
        pl.kernel,
        out_type=jax.ShapeDtypeStruct((_B, _S, _D), jnp.float32),
        mesh=mesh,
        scratch_types=(
            [pltpu.VMEM((_P, _D), jnp.float32)]
            + [pltpu.VMEM((_B, _RSUB, _D), jnp.float32)
               for _ in range(_NSLOT)]
            + [pltpu.SemaphoreType.DMA for _ in range(2 * _NSLOT + 2)]
        ),
    )(_body)
    return f(we, pos)


def kernel(input_ids, word_embeddings, pos_table):
    del input_ids
    return _sc_add(word_embeddings, pos_table)


# speedup vs baseline: 1.0386x; 1.0009x over previous
import functools

import jax
import jax.numpy as jnp
from jax import lax
from jax.experimental import pallas as pl
from jax.experimental.pallas import tpu as pltpu
from jax.experimental.pallas import tpu_sc as plsc

_B, _S, _D = 4, 8192, 128
_NC, _NS, _L = 2, 16, 16
_NW = _NC * _NS
_P = _S // _NW
_RSUB = 32
_NSUB = _P // _RSUB
_NSLOT = 5
_LEAD = 4


def _body(we_hbm, pos_hbm, out_hbm, pos_v, *scratch):
    buf = scratch[0:_NSLOT]
    psem = scratch[_NSLOT:_NSLOT + 2]
    ld = scratch[_NSLOT + 2:_NSLOT + 2 + _NSLOT]
    st = scratch[_NSLOT + 2 + _NSLOT:_NSLOT + 2 + 2 * _NSLOT]

    wid = lax.axis_index("s") * _NC + lax.axis_index("c")
    prow = wid * _P

    _PHEAD = _LEAD * _RSUB
    pos_a = pltpu.make_async_copy(
        pos_hbm.at[pl.ds(prow, _PHEAD)], pos_v.at[pl.ds(0, _PHEAD)], psem[0])
    pos_b = pltpu.make_async_copy(
        pos_hbm.at[pl.ds(prow + _PHEAD, _P - _PHEAD)],
        pos_v.at[pl.ds(_PHEAD, _P - _PHEAD)], psem[1])
    pos_a.start()
    pos_b.start()

    def fire_loads(t):
        slot = t % _NSLOT
        r0 = prow + t * _RSUB
        pltpu.async_copy(
            we_hbm.at[:, pl.ds(r0, _RSUB)], buf[slot], ld[slot])

    def wait_loads(t):
        slot = t % _NSLOT
        r0 = prow + t * _RSUB
        pltpu.make_async_copy(
            we_hbm.at[:, pl.ds(r0, _RSUB)], buf[slot], ld[slot]).wait()

    def fire_stores(t):
        slot = t % _NSLOT
        r0 = prow + t * _RSUB
        pltpu.async_copy(
            buf[slot], out_hbm.at[:, pl.ds(r0, _RSUB)], st[slot])

    def wait_stores(t):
        slot = t % _NSLOT
        r0 = prow + t * _RSUB
        pltpu.make_async_copy(
            buf[slot], out_hbm.at[:, pl.ds(r0, _RSUB)], st[slot]).wait()

    for t in range(_LEAD):
        fire_loads(t)
    pos_a.wait()

    for t in range(_NSUB):
        slot = t % _NSLOT
        wait_loads(t)
        if t == _LEAD:
            pos_b.wait()

        def row(i, carry):
            for j in range(_D // _L):
                sl = pl.ds(j * _L, _L)
                p = pos_v[t * _RSUB + i, sl]
                for b in range(_B):
                    buf[slot][b, i, sl] = buf[slot][b, i, sl] + p
            return carry

        lax.fori_loop(0, _RSUB, row, 0)
        fire_stores(t)
        if t + _LEAD < _NSUB:
            if t + _LEAD >= _NSLOT:
                wait_stores(t + _LEAD - _NSLOT)
            fire_loads(t + _LEAD)

    for t in range(_NSUB - _NSLOT, _NSUB):
        if t >= 0:
            wait_stores(t)


@jax.jit
def _sc_add(we, pos):
    mesh = plsc.VectorSubcoreMesh(core_axis_name="c", subcore_axis_name="s")
    f = functools.partial(
        pl.kernel,
        out_type=jax.ShapeDtypeStruct((_B, _S, _D), jnp.float32),
        mesh=mesh,
        scratch_types=(
            [pltpu.VMEM((_P, _D), jnp.float32)]
            + [pltpu.VMEM((_B, _RSUB, _D), jnp.float32)
               for _ in range(_NSLOT)]
            + [pltpu.SemaphoreType.DMA for _ in range(2 * _NSLOT + 2)]
        ),
    )(_body)
    return f(we, pos)


def kernel(input_ids, word_embeddings, pos_table):
    del input_ids
    return _sc_add(word_embeddings, pos_table)
